# baseline (speedup 1.0000x reference)
import functools

import jax
import jax.numpy as jnp
from jax import lax
from jax.experimental import pallas as pl
from jax.experimental.pallas import tpu as pltpu
from jax.experimental.pallas import tpu_sc as plsc

N = 10000
D = 128
H = 128
V = 10000
E = 320000

NC = 2
NS = 16
NW = NC * NS
CH = 256
EPW = E // NW
NCH = 40
EPW_PAD = NCH * CH
NPAD = 10240
RPT = NPAD // NS
TRASH = N + 16

_STRIPES = []
_o = 0
while _o < RPT:
    _STRIPES.append((_o, min(CH, RPT - _o)))
    _o += min(CH, RPT - _o)


def _hamilton(kernel_w):
    r, i, j, k = jnp.split(kernel_w, 4, axis=1)
    r2 = jnp.concatenate([r, -i, -j, -k], axis=0)
    i2 = jnp.concatenate([i, r, -k, j], axis=0)
    j2 = jnp.concatenate([j, k, r, -i], axis=0)
    k2 = jnp.concatenate([k, -j, i, r], axis=0)
    return jnp.concatenate([r2, i2, j2, k2], axis=1)


_sc_mesh = plsc.VectorSubcoreMesh(core_axis_name="c", subcore_axis_name="s")


@functools.partial(
    pl.kernel,
    out_type=jax.ShapeDtypeStruct((NC, NPAD, H), jnp.float32),
    mesh=_sc_mesh,
    scratch_types=[
        pltpu.VMEM((CH,), jnp.int32),
        pltpu.VMEM((CH,), jnp.int32),
        pltpu.VMEM((CH, H), jnp.float32),
        pltpu.VMEM_SHARED((NPAD, H), jnp.float32),
        pltpu.SemaphoreType.DMA,
    ],
)
def _seg_sum(src_hbm, dst_hbm, sup_hbm, out_hbm, sidx, didx, rows, acc, gsem):
    c = lax.axis_index("c")
    s = lax.axis_index("s")
    w = c * NS + s

    def _zrow(r, carry):
        for q in range(H // 16):
            rows[r, pl.ds(q * 16, 16)] = jnp.zeros((16,), jnp.float32)
        return carry

    lax.fori_loop(0, CH, _zrow, 0)
    for off, sz in _STRIPES:
        pltpu.sync_copy(rows.at[pl.ds(0, sz)],
                        acc.at[pl.ds(s * RPT + off, sz)])
    plsc.subcore_barrier()

    def _chunk(i, carry):
        pltpu.sync_copy(src_hbm.at[w, i], sidx)
        pltpu.sync_copy(dst_hbm.at[w, i], didx)
        pltpu.async_copy(sup_hbm.at[sidx], rows, gsem).wait()
        pltpu.sync_copy(rows, acc.at[didx], add=True)
        return carry

    lax.fori_loop(0, NCH, _chunk, 0)
    plsc.subcore_barrier()

    for off, sz in _STRIPES:
        span = pl.ds(s * RPT + off, sz)
        pltpu.sync_copy(acc.at[span], rows.at[pl.ds(0, sz)])
        pltpu.sync_copy(rows.at[pl.ds(0, sz)], out_hbm.at[c, span])


def _mm_body(x_ref, w_ref, o_ref):
    o_ref[...] = jnp.dot(x_ref[...], w_ref[...],
                         preferred_element_type=jnp.float32)


def _support1(x, w):
    return pl.pallas_call(
        _mm_body,
        grid=(5,),
        in_specs=[pl.BlockSpec((2000, D), lambda i: (i, 0)),
                  pl.BlockSpec((D, H), lambda i: (0, 0))],
        out_specs=pl.BlockSpec((2000, H), lambda i: (i, 0)),
        out_shape=jax.ShapeDtypeStruct((N, H), jnp.float32),
    )(x, w)


def _l2_body(agg_ref, w_ref, h1_ref, s2_ref):
    h1 = jnp.tanh(agg_ref[0] + agg_ref[1])
    h1_ref[...] = h1
    s2_ref[...] = jnp.dot(h1, w_ref[...], preferred_element_type=jnp.float32)


def _layer2(aggp, w):
    return pl.pallas_call(
        _l2_body,
        grid=(5,),
        in_specs=[pl.BlockSpec((NC, 2000, H), lambda i: (0, i, 0)),
                  pl.BlockSpec((H, H), lambda i: (0, 0))],
        out_specs=[pl.BlockSpec((2000, H), lambda i: (i, 0))] * 2,
        out_shape=[jax.ShapeDtypeStruct((N, H), jnp.float32)] * 2,
    )(aggp, w)


BM = 512
BV = 2048


def _head_body(agg_ref, h1_ref, w1_ref, w2_ref, b_ref, o_ref):
    h2 = jnp.tanh(agg_ref[0] + agg_ref[1])
    dn = (((1,), (1,)), ((), ()))
    acc = lax.dot_general(h1_ref[...], w1_ref[...], dn,
                          preferred_element_type=jnp.float32)
    acc = acc + lax.dot_general(h2, w2_ref[...], dn,
                                preferred_element_type=jnp.float32)
    o_ref[...] = acc + b_ref[...]


def _head(aggp, h1, w1, w2, b):
    gm = -(-N // BM)
    gv = -(-V // BV)
    return pl.pallas_call(
        _head_body,
        grid=(gm, gv),
        in_specs=[
            pl.BlockSpec((NC, BM, H), lambda i, j: (0, i, 0)),
            pl.BlockSpec((BM, H), lambda i, j: (i, 0)),
            pl.BlockSpec((BV, H), lambda i, j: (j, 0)),
            pl.BlockSpec((BV, H), lambda i, j: (j, 0)),
            pl.BlockSpec((1, BV), lambda i, j: (0, j)),
        ],
        out_specs=pl.BlockSpec((BM, BV), lambda i, j: (i, j)),
        out_shape=jax.ShapeDtypeStruct((N, V), jnp.float32),
    )(aggp, h1, w1, w2, b)


def kernel(Adj_block, X_concat, idx_nodes, W1, W2, sm_weight, sm_bias):
    h1w = _hamilton(W1)
    h2w = _hamilton(W2)

    dst = Adj_block[0].reshape(NW, EPW)
    src = Adj_block[1].reshape(NW, EPW)
    pad = NCH * CH - EPW
    src3 = jnp.concatenate(
        [src, jnp.zeros((NW, pad), jnp.int32)], axis=1).reshape(NW, NCH, CH)
    dst3 = jnp.concatenate(
        [dst, jnp.full((NW, pad), TRASH, jnp.int32)], axis=1).reshape(NW, NCH, CH)
    sup1 = _support1(X_concat, h1w)
    aggp1 = _seg_sum(src3, dst3, sup1)
    h1, sup2 = _layer2(aggp1, h2w)
    aggp2 = _seg_sum(src3, dst3, sup2)
    return _head(aggp2, h1, sm_weight[:, :H], sm_weight[:, H:],
                 sm_bias.reshape(1, V))


# speedup vs baseline: 1.2270x; 1.2270x over previous
import functools

import jax
import jax.numpy as jnp
from jax import lax
from jax.experimental import pallas as pl
from jax.experimental.pallas import tpu as pltpu
from jax.experimental.pallas import tpu_sc as plsc

N = 10000
D = 128
H = 128
V = 10000
E = 320000

NC = 2
NS = 16
NW = NC * NS
CH = 128
EPW = E // NW
NCH = -(-EPW // CH)
EPW_PAD = NCH * CH
NPAD = 10240
RPT = NPAD // NS
TRASH = N + 16

_STRIPES = []
_o = 0
while _o < RPT:
    _STRIPES.append((_o, min(CH, RPT - _o)))
    _o += min(CH, RPT - _o)


def _hamilton(kernel_w):
    r, i, j, k = jnp.split(kernel_w, 4, axis=1)
    r2 = jnp.concatenate([r, -i, -j, -k], axis=0)
    i2 = jnp.concatenate([i, r, -k, j], axis=0)
    j2 = jnp.concatenate([j, k, r, -i], axis=0)
    k2 = jnp.concatenate([k, -j, i, r], axis=0)
    return jnp.concatenate([r2, i2, j2, k2], axis=1)


_sc_mesh = plsc.VectorSubcoreMesh(core_axis_name="c", subcore_axis_name="s")


@functools.partial(
    pl.kernel,
    out_type=jax.ShapeDtypeStruct((NC, NPAD, H), jnp.float32),
    mesh=_sc_mesh,
    scratch_types=[
        pltpu.VMEM((CH,), jnp.int32),
        pltpu.VMEM((CH,), jnp.int32),
        pltpu.VMEM((CH, H), jnp.float32),
        pltpu.VMEM_SHARED((NPAD, H), jnp.float32),
        pltpu.SemaphoreType.DMA,
    ],
)
def _seg_sum(src_hbm, dst_hbm, sup_hbm, out_hbm, sidx, didx, rows, acc, gsem):
    c = lax.axis_index("c")
    s = lax.axis_index("s")
    w = c * NS + s

    def _zrow(r, carry):
        for q in range(H // 16):
            rows[r, pl.ds(q * 16, 16)] = jnp.zeros((16,), jnp.float32)
        return carry

    lax.fori_loop(0, CH, _zrow, 0)
    for off, sz in _STRIPES:
        pltpu.sync_copy(rows.at[pl.ds(0, sz)],
                        acc.at[pl.ds(s * RPT + off, sz)])
    plsc.subcore_barrier()

    def _chunk(i, carry):
        pltpu.sync_copy(src_hbm.at[w, i], sidx)
        pltpu.sync_copy(dst_hbm.at[w, i], didx)
        pltpu.async_copy(sup_hbm.at[sidx], rows, gsem).wait()
        pltpu.sync_copy(rows, acc.at[didx], add=True)
        return carry

    lax.fori_loop(0, NCH, _chunk, 0)
    plsc.subcore_barrier()

    for off, sz in _STRIPES:
        span = pl.ds(s * RPT + off, sz)
        pltpu.sync_copy(acc.at[span], rows.at[pl.ds(0, sz)])
        pltpu.sync_copy(rows.at[pl.ds(0, sz)], out_hbm.at[c, span])


def _mm_body(x_ref, w_ref, o_ref):
    o_ref[...] = jnp.dot(x_ref[...], w_ref[...],
                         preferred_element_type=jnp.float32)


def _support1(x, w):
    return pl.pallas_call(
        _mm_body,
        grid=(5,),
        in_specs=[pl.BlockSpec((2000, D), lambda i: (i, 0)),
                  pl.BlockSpec((D, H), lambda i: (0, 0))],
        out_specs=pl.BlockSpec((2000, H), lambda i: (i, 0)),
        out_shape=jax.ShapeDtypeStruct((N, H), jnp.float32),
    )(x, w)


def _l2_body(agg_ref, w_ref, h1_ref, s2_ref):
    h1 = jnp.tanh(agg_ref[0] + agg_ref[1])
    h1_ref[...] = h1.astype(jnp.bfloat16)
    s2_ref[...] = jnp.dot(h1, w_ref[...], preferred_element_type=jnp.float32)


def _layer2(aggp, w):
    return pl.pallas_call(
        _l2_body,
        grid=(5,),
        in_specs=[pl.BlockSpec((NC, 2000, H), lambda i: (0, i, 0)),
                  pl.BlockSpec((H, H), lambda i: (0, 0))],
        out_specs=[pl.BlockSpec((2000, H), lambda i: (i, 0))] * 2,
        out_shape=[jax.ShapeDtypeStruct((N, H), jnp.bfloat16),
                   jax.ShapeDtypeStruct((N, H), jnp.float32)],
    )(aggp, w)


BM = 512
BV = 2048


def _head_body(agg_ref, h1_ref, w1_ref, w2_ref, b_ref, o_ref):
    h2 = jnp.tanh(agg_ref[0] + agg_ref[1]).astype(jnp.bfloat16)
    dn = (((1,), (1,)), ((), ()))
    acc = lax.dot_general(h1_ref[...], w1_ref[...], dn,
                          preferred_element_type=jnp.float32)
    acc = acc + lax.dot_general(h2, w2_ref[...], dn,
                                preferred_element_type=jnp.float32)
    o_ref[...] = acc + b_ref[...]


def _head(aggp, h1, w1, w2, b):
    gm = -(-N // BM)
    gv = -(-V // BV)
    return pl.pallas_call(
        _head_body,
        grid=(gm, gv),
        in_specs=[
            pl.BlockSpec((NC, BM, H), lambda i, j: (0, i, 0)),
            pl.BlockSpec((BM, H), lambda i, j: (i, 0)),
            pl.BlockSpec((BV, H), lambda i, j: (j, 0)),
            pl.BlockSpec((BV, H), lambda i, j: (j, 0)),
            pl.BlockSpec((1, BV), lambda i, j: (0, j)),
        ],
        out_specs=pl.BlockSpec((BM, BV), lambda i, j: (i, j)),
        out_shape=jax.ShapeDtypeStruct((N, V), jnp.float32),
    )(aggp, h1, w1, w2, b)


def kernel(Adj_block, X_concat, idx_nodes, W1, W2, sm_weight, sm_bias):
    h1w = _hamilton(W1)
    h2w = _hamilton(W2)

    dst = Adj_block[0].reshape(NW, EPW)
    src = Adj_block[1].reshape(NW, EPW)
    pad = NCH * CH - EPW
    src3 = jnp.concatenate(
        [src, jnp.zeros((NW, pad), jnp.int32)], axis=1).reshape(NW, NCH, CH)
    dst3 = jnp.concatenate(
        [dst, jnp.full((NW, pad), TRASH, jnp.int32)], axis=1).reshape(NW, NCH, CH)
    sup1 = _support1(X_concat, h1w)
    aggp1 = _seg_sum(src3, dst3, sup1)
    h1, sup2 = _layer2(aggp1, h2w)
    aggp2 = _seg_sum(src3, dst3, sup2)
    w1bf = sm_weight[:, :H].astype(jnp.bfloat16)
    w2bf = sm_weight[:, H:].astype(jnp.bfloat16)
    return _head(aggp2, h1, w1bf, w2bf, sm_bias.reshape(1, V))
